# baseline (speedup 1.0000x reference)
import functools

import jax
import jax.numpy as jnp
from jax import lax
from jax.experimental import pallas as pl
from jax.experimental.pallas import tpu as pltpu
from jax.experimental.pallas import tpu_sc as plsc

TOK = 1024
NUM_CODES = 8192
DIM = 64


def _vq_argmin_body(x_ref, e_ref, x2_ref, e2_ref, idx_ref, tab_ref):
    @pl.when(pl.program_id(0) == 0)
    def _():
        tab_ref[:, :DIM] = e_ref[...].T

    xb = x_ref[0] * -2.0
    xe = lax.dot_general(
        xb, e_ref[...], (((0,), (0,)), ((), ())),
        preferred_element_type=jnp.float32,
    )
    x2 = x2_ref[0, 0, :][:, None]
    s = (x2 + xe) + e2_ref[...]
    idx_ref[0, 0, :] = jnp.argmin(s, axis=1).astype(jnp.int32)


GATHER_D = 128


def _gather_sc(table, idx):
    n = idx.shape[0]
    info = plsc.get_sparse_core_info()
    nw = info.num_cores * info.num_subcores
    b_per_w = n // nw
    mesh = plsc.VectorSubcoreMesh(core_axis_name="c", subcore_axis_name="s")

    @functools.partial(
        pl.kernel,
        mesh=mesh,
        out_type=jax.ShapeDtypeStruct((n, GATHER_D), jnp.float32),
        scratch_types=[
            pltpu.VMEM((b_per_w,), jnp.int32),
            pltpu.VMEM((b_per_w, GATHER_D), jnp.float32),
            pltpu.SemaphoreType.DMA,
        ],
    )
    def k(table_hbm, idx_hbm, out_hbm, idx_v, rows_v, sem):
        wid = lax.axis_index("s") * info.num_cores + lax.axis_index("c")
        base = wid * b_per_w
        pltpu.sync_copy(idx_hbm.at[pl.ds(base, b_per_w)], idx_v)
        pltpu.async_copy(table_hbm.at[idx_v], rows_v, sem).wait()
        pltpu.sync_copy(rows_v, out_hbm.at[pl.ds(base, b_per_w)])

    return k(table, idx)


def _argmin_tc(x3, e_i_ts, x2, e2):
    Bh, C, HW = x3.shape
    n = Bh * HW
    grid = n // TOK
    pb = HW // TOK
    out = pl.pallas_call(
        _vq_argmin_body,
        grid=(grid,),
        in_specs=[
            pl.BlockSpec((1, C, TOK), lambda i: (i // pb, 0, i % pb)),
            pl.BlockSpec((C, NUM_CODES), lambda i: (0, 0)),
            pl.BlockSpec((1, 1, TOK), lambda i: (i, 0, 0)),
            pl.BlockSpec((1, NUM_CODES), lambda i: (0, 0)),
        ],
        out_specs=[
            pl.BlockSpec((1, 1, TOK), lambda i: (i, 0, 0)),
            pl.BlockSpec((NUM_CODES, GATHER_D), lambda i: (0, 0)),
        ],
        out_shape=[
            jax.ShapeDtypeStruct((grid, 1, TOK), jnp.int32),
            jax.ShapeDtypeStruct((NUM_CODES, GATHER_D), jnp.float32),
        ],
    )(x3, e_i_ts, x2.reshape(grid, 1, TOK), e2)
    idx, tab = out
    return idx.reshape(n), tab


def kernel(x, e_i_ts):
    B, C, H, W = x.shape
    x3 = x.reshape(B, C, H * W)
    flat_x = jnp.transpose(x, (0, 2, 3, 1)).reshape(-1, C)
    x2 = (flat_x ** 2).sum(axis=1)
    e2 = (e_i_ts ** 2).sum(axis=0, keepdims=True)
    flat_idx, table = _argmin_tc(x3, e_i_ts, x2, e2)
    quant = _gather_sc(table, flat_idx)
    out = quant.reshape(B, H, W, GATHER_D)[..., :C].transpose(0, 3, 1, 2)
    return out
